# baseline (speedup 1.0000x reference)
import jax
import jax.numpy as jnp
from jax.experimental import pallas as pl

_BLOCK = 5000


def _body(x_ref, t_ref, pred_ref, datan_ref, taskn_ref):
    t = t_ref[...]
    tn = t / jnp.maximum(
        jnp.sqrt(jnp.sum(t * t, axis=1, keepdims=True)), 1e-12)

    @pl.when(pl.program_id(0) == 0)
    def _():
        taskn_ref[...] = tn

    x = x_ref[...]
    xn = x / jnp.maximum(
        jnp.sqrt(jnp.sum(x * x, axis=1, keepdims=True)), 1e-12)
    datan_ref[...] = xn
    pred_ref[...] = jax.lax.dot_general(
        xn, tn,
        dimension_numbers=(((1,), (1,)), ((), ())),
        preferred_element_type=jnp.float32)


def kernel(graph_feature, graph_targets_id_batch, graph_targets_id,
           graph_targets_value, task_emb):
    n, d = graph_feature.shape
    k = task_emb.shape[0]
    blk = _BLOCK if n % _BLOCK == 0 else n
    pred, data_n, task_n = pl.pallas_call(
        _body,
        grid=(n // blk,),
        in_specs=[
            pl.BlockSpec((blk, d), lambda i: (i, 0)),
            pl.BlockSpec((k, d), lambda i: (0, 0)),
        ],
        out_specs=[
            pl.BlockSpec((blk, k), lambda i: (i, 0)),
            pl.BlockSpec((blk, d), lambda i: (i, 0)),
            pl.BlockSpec((k, d), lambda i: (0, 0)),
        ],
        out_shape=[
            jax.ShapeDtypeStruct((n, k), jnp.float32),
            jax.ShapeDtypeStruct((n, d), jnp.float32),
            jax.ShapeDtypeStruct((k, d), jnp.float32),
        ],
    )(graph_feature, task_emb)
    return (pred, data_n, task_n)


# speedup vs baseline: 1.0808x; 1.0808x over previous
import jax
import jax.numpy as jnp
from jax.experimental import pallas as pl

_BLOCK = 18000


def _body(x_ref, t_ref, pred_ref, datan_ref, taskn_ref):
    t = t_ref[...]
    tn = t / jnp.maximum(
        jnp.sqrt(jnp.sum(t * t, axis=1, keepdims=True)), 1e-12)

    @pl.when(pl.program_id(0) == 0)
    def _():
        taskn_ref[...] = tn

    x = x_ref[...]
    xn = x / jnp.maximum(
        jnp.sqrt(jnp.sum(x * x, axis=1, keepdims=True)), 1e-12)
    datan_ref[...] = xn
    pred_ref[...] = jax.lax.dot_general(
        xn, tn,
        dimension_numbers=(((1,), (1,)), ((), ())),
        preferred_element_type=jnp.float32)


def kernel(graph_feature, graph_targets_id_batch, graph_targets_id,
           graph_targets_value, task_emb):
    n, d = graph_feature.shape
    k = task_emb.shape[0]
    blk = min(_BLOCK, n)
    pred, data_n, task_n = pl.pallas_call(
        _body,
        grid=(pl.cdiv(n, blk),),
        in_specs=[
            pl.BlockSpec((blk, d), lambda i: (i, 0)),
            pl.BlockSpec((k, d), lambda i: (0, 0)),
        ],
        out_specs=[
            pl.BlockSpec((blk, k), lambda i: (i, 0)),
            pl.BlockSpec((blk, d), lambda i: (i, 0)),
            pl.BlockSpec((k, d), lambda i: (0, 0)),
        ],
        out_shape=[
            jax.ShapeDtypeStruct((n, k), jnp.float32),
            jax.ShapeDtypeStruct((n, d), jnp.float32),
            jax.ShapeDtypeStruct((k, d), jnp.float32),
        ],
    )(graph_feature, task_emb)
    return (pred, data_n, task_n)


# speedup vs baseline: 1.0853x; 1.0041x over previous
import jax
import jax.numpy as jnp
from jax.experimental import pallas as pl

_BLOCK = 16672


def _body(x_ref, t_ref, pred_ref, datan_ref, taskn_ref):
    t = t_ref[...]
    tn = t / jnp.maximum(
        jnp.sqrt(jnp.sum(t * t, axis=1, keepdims=True)), 1e-12)

    @pl.when(pl.program_id(0) == 0)
    def _():
        taskn_ref[...] = tn

    x = x_ref[...]
    xn = x / jnp.maximum(
        jnp.sqrt(jnp.sum(x * x, axis=1, keepdims=True)), 1e-12)
    datan_ref[...] = xn
    pred_ref[...] = jax.lax.dot_general(
        xn, tn,
        dimension_numbers=(((1,), (1,)), ((), ())),
        preferred_element_type=jnp.float32)


def kernel(graph_feature, graph_targets_id_batch, graph_targets_id,
           graph_targets_value, task_emb):
    n, d = graph_feature.shape
    k = task_emb.shape[0]
    blk = min(_BLOCK, n)
    pred, data_n, task_n = pl.pallas_call(
        _body,
        grid=(pl.cdiv(n, blk),),
        in_specs=[
            pl.BlockSpec((blk, d), lambda i: (i, 0)),
            pl.BlockSpec((k, d), lambda i: (0, 0)),
        ],
        out_specs=[
            pl.BlockSpec((blk, k), lambda i: (i, 0)),
            pl.BlockSpec((blk, d), lambda i: (i, 0)),
            pl.BlockSpec((k, d), lambda i: (0, 0)),
        ],
        out_shape=[
            jax.ShapeDtypeStruct((n, k), jnp.float32),
            jax.ShapeDtypeStruct((n, d), jnp.float32),
            jax.ShapeDtypeStruct((k, d), jnp.float32),
        ],
    )(graph_feature, task_emb)
    return (pred, data_n, task_n)
